# baseline (speedup 1.0000x reference)
import functools

import jax
import jax.numpy as jnp
from jax import lax
from jax.experimental import pallas as pl
from jax.experimental.pallas import tpu as pltpu
from jax.experimental.pallas import tpu_sc as plsc

_LAMBDA = 20.0
_B, _T, _H, _W, _N = 8, 16, 256, 256, 512
_R = _B * _T
_NC, _NS, _L = 2, 16, 16
_NW = _NC * _NS
_ROWS_PER_W = _R // _NW
_CHUNKS = _N // _L
_HH = _H // 2

_mesh = plsc.VectorSubcoreMesh(
    core_axis_name="c", subcore_axis_name="s", num_cores=_NC, num_subcores=_NS
)


def _point_loss_sc_body(coords_hbm, pred_hbm, svals_hbm, out_hbm,
                        c2_v, coords_v, sv_v, bufA, bufB, bufC, acc_v, acc2_v,
                        idx0_v, part_sh, red_v, semA, semB, semC):
    cid = lax.axis_index("c")
    sid = lax.axis_index("s")
    wid = cid * _NS + sid
    base_row = wid * _ROWS_PER_W

    bufs = (bufA, bufB, bufC)
    sems = (semA, semB, semC)
    n_stages = _ROWS_PER_W * 2

    b0 = base_row // _T
    t0 = base_row % _T

    def start(t):
        j, h = divmod(t, 2)
        return pltpu.async_copy(
            pred_hbm.at[b0, t0 + j, 0, pl.ds(_HH * h, _HH), :],
            bufs[t % 3], sems[t % 3])

    cp = {0: start(0), 1: start(1)}
    pltpu.sync_copy(svals_hbm.at[b0, pl.ds(t0, _ROWS_PER_W), :], sv_v)

    zeros = jnp.zeros((_L,), jnp.int32)
    ones = zeros + 1
    for k in range(_N // 128):
        pltpu.sync_copy(coords_hbm.at[pl.ds(k * 128, 128), :], c2_v)
        for i in range(128 // _L):
            pidx = lax.iota(jnp.int32, _L) + (i * _L)
            coords_v[pl.ds(k * 128 + i * _L, _L)] = plsc.load_gather(
                c2_v, [pidx, zeros])
            coords_v[pl.ds(_N + k * 128 + i * _L, _L)] = plsc.load_gather(
                c2_v, [pidx, ones])
    acc = jnp.zeros((_L,), jnp.float32)
    for t in range(n_stages):
        if t + 2 < n_stages:
            cp[t + 2] = start(t + 2)
        cp[t].wait()
        j, h = divmod(t, 2)
        buf = bufs[t % 3]
        for i in range(_CHUNKS):
            r = coords_v[pl.ds(i * _L, _L)]
            c = coords_v[pl.ds(_N + i * _L, _L)]
            mask = (r >> 7) == h
            g = plsc.load_gather(buf, [r & (_HH - 1), c])
            s = sv_v[j, pl.ds(i * _L, _L)]
            d = g - s
            acc = acc + jnp.where(mask, d * d, jnp.zeros((_L,), jnp.float32))

    @pl.when(sid == 0)
    def _init():
        acc2_v[0, :] = jnp.zeros((_L,), jnp.float32)
        pltpu.sync_copy(acc2_v, part_sh)
    plsc.subcore_barrier()

    idx0_v[...] = jnp.zeros((_L,), jnp.int32)
    acc2_v[0, :] = acc
    pltpu.sync_copy(acc2_v, part_sh.at[idx0_v.at[pl.ds(0, 1)]], add=True)
    plsc.subcore_barrier()

    @pl.when(sid == 0)
    def _reduce():
        pltpu.sync_copy(part_sh, red_v)
        tot = red_v[0, :]
        lanes = lax.iota(jnp.int32, _L)
        dnums = lax.GatherDimensionNumbers(
            offset_dims=(), collapsed_slice_dims=(0,), start_index_map=(0,))
        for sh in (8, 4, 2, 1):
            perm = jnp.bitwise_xor(lanes, sh)
            shuf = lax.gather(tot, perm[:, None], dnums, (1,),
                              mode=lax.GatherScatterMode.PROMISE_IN_BOUNDS)
            tot = tot + shuf
        tot = tot * (_LAMBDA / (_B * _T * _N))
        acc_v[...] = tot
        pltpu.sync_copy(acc_v, out_hbm.at[pl.ds(cid * _L, _L)])


_point_loss_sc = functools.partial(
    pl.kernel,
    out_type=jax.ShapeDtypeStruct((_NC * _L,), jnp.float32),
    mesh=_mesh,
    scratch_types=[
        pltpu.VMEM((128, 2), jnp.int32),
        pltpu.VMEM((2 * _N,), jnp.int32),
        pltpu.VMEM((_ROWS_PER_W, _N), jnp.float32),
        pltpu.VMEM((_HH, _W), jnp.float32),
        pltpu.VMEM((_HH, _W), jnp.float32),
        pltpu.VMEM((_HH, _W), jnp.float32),
        pltpu.VMEM((_L,), jnp.float32),
        pltpu.VMEM((1, _L), jnp.float32),
        pltpu.VMEM((_L,), jnp.int32),
        pltpu.VMEM_SHARED((1, _L), jnp.float32),
        pltpu.VMEM((1, _L), jnp.float32),
        pltpu.SemaphoreType.DMA,
        pltpu.SemaphoreType.DMA,
        pltpu.SemaphoreType.DMA,
    ],
    compiler_params=pltpu.CompilerParams(
        use_tc_tiling_on_sc=True, needs_layout_passes=False),
)(_point_loss_sc_body)


def kernel(pred, s_coords, s_values):
    out = _point_loss_sc(s_coords.astype(jnp.int32), pred, s_values)
    return out[0] + out[_L]


# speedup vs baseline: 1.2738x; 1.2738x over previous
import functools

import jax
import jax.numpy as jnp
from jax import lax
from jax.experimental import pallas as pl
from jax.experimental.pallas import tpu as pltpu
from jax.experimental.pallas import tpu_sc as plsc

_LAMBDA = 20.0
_B, _T, _H, _W, _N = 8, 16, 256, 256, 512
_R = _B * _T
_NC, _NS, _L = 2, 16, 16
_NW = _NC * _NS
_ROWS_PER_W = _R // _NW
_CHUNKS = _N // _L
_HH = _H // 2

_mesh = plsc.VectorSubcoreMesh(
    core_axis_name="c", subcore_axis_name="s", num_cores=_NC, num_subcores=_NS
)


def _point_loss_sc_body(coords_hbm, pred_hbm, svals_hbm, out_hbm,
                        coords_v, sv_v, bufA, bufB, bufC, acc_v, acc2_v,
                        idx0_v, part_sh, red_v, semA, semB, semC):
    cid = lax.axis_index("c")
    sid = lax.axis_index("s")
    wid = cid * _NS + sid
    base_row = wid * _ROWS_PER_W

    bufs = (bufA, bufB, bufC)
    sems = (semA, semB, semC)
    n_stages = _ROWS_PER_W * 2

    b0 = base_row // _T
    t0 = base_row % _T

    def start(t):
        j, h = divmod(t, 2)
        return pltpu.async_copy(
            pred_hbm.at[b0, t0 + j, 0, pl.ds(_HH * h, _HH), :],
            bufs[t % 3], sems[t % 3])

    cp = {0: start(0), 1: start(1)}
    pltpu.sync_copy(coords_hbm, coords_v)
    pltpu.sync_copy(svals_hbm.at[b0, pl.ds(t0, _ROWS_PER_W), :], sv_v)
    acc = jnp.zeros((_L,), jnp.float32)
    for t in range(n_stages):
        if t + 2 < n_stages:
            cp[t + 2] = start(t + 2)
        cp[t].wait()
        j, h = divmod(t, 2)
        buf = bufs[t % 3]
        for i in range(_CHUNKS):
            r = coords_v[pl.ds(i * _L, _L)]
            c = coords_v[pl.ds(_N + i * _L, _L)]
            mask = (r >> 7) == h
            g = plsc.load_gather(buf, [r & (_HH - 1), c])
            s = sv_v[j, pl.ds(i * _L, _L)]
            d = g - s
            acc = acc + jnp.where(mask, d * d, jnp.zeros((_L,), jnp.float32))

    @pl.when(sid == 0)
    def _init():
        acc2_v[0, :] = jnp.zeros((_L,), jnp.float32)
        pltpu.sync_copy(acc2_v, part_sh)
    plsc.subcore_barrier()

    idx0_v[...] = jnp.zeros((_L,), jnp.int32)
    acc2_v[0, :] = acc
    pltpu.sync_copy(acc2_v, part_sh.at[idx0_v.at[pl.ds(0, 1)]], add=True)
    plsc.subcore_barrier()

    @pl.when(sid == 0)
    def _reduce():
        pltpu.sync_copy(part_sh, red_v)
        tot = red_v[0, :]
        lanes = lax.iota(jnp.int32, _L)
        dnums = lax.GatherDimensionNumbers(
            offset_dims=(), collapsed_slice_dims=(0,), start_index_map=(0,))
        for sh in (8, 4, 2, 1):
            perm = jnp.bitwise_xor(lanes, sh)
            shuf = lax.gather(tot, perm[:, None], dnums, (1,),
                              mode=lax.GatherScatterMode.PROMISE_IN_BOUNDS)
            tot = tot + shuf
        tot = tot * (_LAMBDA / (_B * _T * _N))
        acc_v[...] = tot
        pltpu.sync_copy(acc_v, out_hbm.at[pl.ds(cid * _L, _L)])


_point_loss_sc = functools.partial(
    pl.kernel,
    out_type=jax.ShapeDtypeStruct((_NC * _L,), jnp.float32),
    mesh=_mesh,
    scratch_types=[
        pltpu.VMEM((2 * _N,), jnp.int32),
        pltpu.VMEM((_ROWS_PER_W, _N), jnp.float32),
        pltpu.VMEM((_HH, _W), jnp.float32),
        pltpu.VMEM((_HH, _W), jnp.float32),
        pltpu.VMEM((_HH, _W), jnp.float32),
        pltpu.VMEM((_L,), jnp.float32),
        pltpu.VMEM((1, _L), jnp.float32),
        pltpu.VMEM((_L,), jnp.int32),
        pltpu.VMEM_SHARED((1, _L), jnp.float32),
        pltpu.VMEM((1, _L), jnp.float32),
        pltpu.SemaphoreType.DMA,
        pltpu.SemaphoreType.DMA,
        pltpu.SemaphoreType.DMA,
    ],
    compiler_params=pltpu.CompilerParams(
        use_tc_tiling_on_sc=True, needs_layout_passes=False),
)(_point_loss_sc_body)


def kernel(pred, s_coords, s_values):
    sc = s_coords.astype(jnp.int32)
    coords_flat = jnp.concatenate([sc[:, 0], sc[:, 1]])
    out = _point_loss_sc(coords_flat, pred, s_values)
    return out[0] + out[_L]


# speedup vs baseline: 1.2935x; 1.0155x over previous
import functools

import jax
import jax.numpy as jnp
from jax import lax
from jax.experimental import pallas as pl
from jax.experimental.pallas import tpu as pltpu
from jax.experimental.pallas import tpu_sc as plsc

_LAMBDA = 20.0
_B, _T, _H, _W, _N = 8, 16, 256, 256, 512
_R = _B * _T
_NC, _NS, _L = 2, 16, 16
_NW = _NC * _NS
_ROWS_PER_W = _R // _NW
_CHUNKS = _N // _L
_HH = _H // 2

_mesh = plsc.VectorSubcoreMesh(
    core_axis_name="c", subcore_axis_name="s", num_cores=_NC, num_subcores=_NS
)


def _point_loss_sc_body(coords_hbm, pred_hbm, svals_hbm, out_hbm,
                        coords_v, sv_v, bufA, bufB, bufC, acc_v, acc2_v,
                        idx0_v, part_sh, red_v, semA, semB, semC):
    cid = lax.axis_index("c")
    sid = lax.axis_index("s")
    wid = cid * _NS + sid
    base_row = wid * _ROWS_PER_W

    bufs = (bufA, bufB, bufC)
    sems = (semA, semB, semC)
    n_stages = _ROWS_PER_W * 2

    b0 = base_row // _T
    t0 = base_row % _T

    _Q = _HH // 2

    def start(t):
        j, h = divmod(t, 2)
        return [
            pltpu.async_copy(
                pred_hbm.at[b0, t0 + j, 0, pl.ds(_HH * h + _Q * q, _Q), :],
                bufs[t % 3].at[pl.ds(_Q * q, _Q), :], sems[t % 3])
            for q in range(2)
        ]

    cp = {0: start(0), 1: start(1)}
    pltpu.sync_copy(coords_hbm, coords_v)
    pltpu.sync_copy(svals_hbm.at[b0, pl.ds(t0, _ROWS_PER_W), :], sv_v)
    acc = jnp.zeros((_L,), jnp.float32)
    for t in range(n_stages):
        if t + 2 < n_stages:
            cp[t + 2] = start(t + 2)
        for _c in cp[t]:
            _c.wait()
        j, h = divmod(t, 2)
        buf = bufs[t % 3]
        for i in range(_CHUNKS):
            r = coords_v[pl.ds(i * _L, _L)]
            c = coords_v[pl.ds(_N + i * _L, _L)]
            mask = (r >> 7) == h
            g = plsc.load_gather(buf, [r & (_HH - 1), c])
            s = sv_v[j, pl.ds(i * _L, _L)]
            d = g - s
            acc = acc + jnp.where(mask, d * d, jnp.zeros((_L,), jnp.float32))

    @pl.when(sid == 0)
    def _init():
        acc2_v[0, :] = jnp.zeros((_L,), jnp.float32)
        pltpu.sync_copy(acc2_v, part_sh)
    plsc.subcore_barrier()

    idx0_v[...] = jnp.zeros((_L,), jnp.int32)
    acc2_v[0, :] = acc
    pltpu.sync_copy(acc2_v, part_sh.at[idx0_v.at[pl.ds(0, 1)]], add=True)
    plsc.subcore_barrier()

    @pl.when(sid == 0)
    def _reduce():
        pltpu.sync_copy(part_sh, red_v)
        tot = red_v[0, :]
        lanes = lax.iota(jnp.int32, _L)
        dnums = lax.GatherDimensionNumbers(
            offset_dims=(), collapsed_slice_dims=(0,), start_index_map=(0,))
        for sh in (8, 4, 2, 1):
            perm = jnp.bitwise_xor(lanes, sh)
            shuf = lax.gather(tot, perm[:, None], dnums, (1,),
                              mode=lax.GatherScatterMode.PROMISE_IN_BOUNDS)
            tot = tot + shuf
        tot = tot * (_LAMBDA / (_B * _T * _N))
        acc_v[...] = tot
        pltpu.sync_copy(acc_v, out_hbm.at[pl.ds(cid * _L, _L)])


_point_loss_sc = functools.partial(
    pl.kernel,
    out_type=jax.ShapeDtypeStruct((_NC * _L,), jnp.float32),
    mesh=_mesh,
    scratch_types=[
        pltpu.VMEM((2 * _N,), jnp.int32),
        pltpu.VMEM((_ROWS_PER_W, _N), jnp.float32),
        pltpu.VMEM((_HH, _W), jnp.float32),
        pltpu.VMEM((_HH, _W), jnp.float32),
        pltpu.VMEM((_HH, _W), jnp.float32),
        pltpu.VMEM((_L,), jnp.float32),
        pltpu.VMEM((1, _L), jnp.float32),
        pltpu.VMEM((_L,), jnp.int32),
        pltpu.VMEM_SHARED((1, _L), jnp.float32),
        pltpu.VMEM((1, _L), jnp.float32),
        pltpu.SemaphoreType.DMA,
        pltpu.SemaphoreType.DMA,
        pltpu.SemaphoreType.DMA,
    ],
    compiler_params=pltpu.CompilerParams(
        use_tc_tiling_on_sc=True, needs_layout_passes=False),
)(_point_loss_sc_body)


def kernel(pred, s_coords, s_values):
    sc = s_coords.astype(jnp.int32)
    coords_flat = jnp.concatenate([sc[:, 0], sc[:, 1]])
    out = _point_loss_sc(coords_flat, pred, s_values)
    return out[0] + out[_L]
